# Initial kernel scaffold; baseline (speedup 1.0000x reference)
import jax
import jax.numpy as jnp
from jax.experimental import pallas as pl


def kernel(features, tables, W1, b1, g1, be1, W2, b2, g2, be2, W3, b3, g3, be3, W4, b4):
    raise NotImplementedError("write your pallas kernel here")


# speedup vs baseline: 7.8204x; 7.8204x over previous
import functools

import jax
import jax.numpy as jnp
from jax import lax
from jax.experimental import pallas as pl
from jax.experimental.pallas import tpu as pltpu
from jax.experimental.pallas import tpu_sc as plsc

NUM_FIELDS = 26
VOCAB = 100000
EMBED_DIM = 16
BATCH = 16384
INPUT_DIM = NUM_FIELDS * EMBED_DIM

NC, NS = 2, 16
NW = NC * NS
TOTAL = BATCH * NUM_FIELDS
PER_W = TOTAL // NW
CHUNK = 128
GROUP = 13
NCHUNK = PER_W // CHUNK
NGROUP = NCHUNK // GROUP

@functools.cache
def _make_sc_gather():
    mesh = plsc.VectorSubcoreMesh(
        core_axis_name="c", subcore_axis_name="s", num_cores=NC, num_subcores=NS
    )

    @functools.partial(
        pl.kernel,
        out_type=jax.ShapeDtypeStruct((TOTAL, EMBED_DIM), jnp.float32),
        mesh=mesh,
        scratch_types=[
            pltpu.VMEM((NCHUNK, CHUNK), jnp.int32),
            pltpu.VMEM((GROUP * CHUNK, EMBED_DIM), jnp.float32),
            pltpu.SemaphoreType.DMA,
        ],
        compiler_params=pltpu.CompilerParams(use_tc_tiling_on_sc=False),
    )
    def _sc_gather(table_hbm, idx_hbm, out_hbm, idx_v, rows_v, sem):
        wid = lax.axis_index("s") * NC + lax.axis_index("c")
        pltpu.sync_copy(idx_hbm.at[wid], idx_v)

        def group_body(g, _):
            copies = [
                pltpu.async_copy(
                    table_hbm.at[idx_v.at[g * GROUP + j]],
                    rows_v.at[pl.ds(j * CHUNK, CHUNK)],
                    sem,
                )
                for j in range(GROUP)
            ]
            for c in copies:
                c.wait()
            pltpu.sync_copy(
                rows_v,
                out_hbm.at[pl.ds(wid * PER_W + g * (GROUP * CHUNK), GROUP * CHUNK)],
            )
            return 0

        lax.fori_loop(0, NGROUP, group_body, 0)

    return _sc_gather


BB = 2048
NB = BATCH // BB
NPASS = 4
EPS = 1e-5
SLOPE = 0.01


def _mlp_body(x_ref, w1, b1, g1, be1, w2, b2, g2, be2, w3, b3, g3, be3, w4, b4,
              out_ref, h12, h3, s1, q1, s2, q2, s3, q3):
    p = pl.program_id(0)
    i = pl.program_id(1)
    rows = pl.ds(i * BB, BB)

    def bn_affine(s, q, g, be):
        m = s[...] / BATCH
        v = q[...] / BATCH - m * m
        a = g[...] * lax.rsqrt(v + EPS)
        c = be[...] - m * a
        return a, c

    @pl.when(p == 0)
    def _():
        h = jnp.dot(x_ref[...], w1[...], preferred_element_type=jnp.float32)
        h = h + b1[...]
        h12[rows, :] = h

        @pl.when(i == 0)
        def _():
            s1[...] = jnp.zeros_like(s1)
            q1[...] = jnp.zeros_like(q1)

        s1[...] += jnp.sum(h, axis=0, keepdims=True)
        q1[...] += jnp.sum(h * h, axis=0, keepdims=True)

    @pl.when(p == 1)
    def _():
        a, c = bn_affine(s1, q1, g1, be1)
        z = h12[rows, :] * a + c
        z = jnp.where(z >= 0, z, SLOPE * z)
        h = jnp.dot(z, w2[...], preferred_element_type=jnp.float32) + b2[...]
        h12[rows, :] = h

        @pl.when(i == 0)
        def _():
            s2[...] = jnp.zeros_like(s2)
            q2[...] = jnp.zeros_like(q2)

        s2[...] += jnp.sum(h, axis=0, keepdims=True)
        q2[...] += jnp.sum(h * h, axis=0, keepdims=True)

    @pl.when(p == 2)
    def _():
        a, c = bn_affine(s2, q2, g2, be2)
        z = h12[rows, :] * a + c
        z = jnp.where(z >= 0, z, SLOPE * z)
        h = jnp.dot(z, w3[...], preferred_element_type=jnp.float32) + b3[...]
        h3[rows, :] = h

        @pl.when(i == 0)
        def _():
            s3[...] = jnp.zeros_like(s3)
            q3[...] = jnp.zeros_like(q3)

        s3[...] += jnp.sum(h, axis=0, keepdims=True)
        q3[...] += jnp.sum(h * h, axis=0, keepdims=True)

    @pl.when(p == 3)
    def _():
        a, c = bn_affine(s3, q3, g3, be3)
        z = h3[rows, :] * a + c
        z = jnp.where(z >= 0, z, SLOPE * z)
        logits = jnp.dot(z, w4[...], preferred_element_type=jnp.float32)
        out_ref[...] = logits[:, 0] + b4[0]


def _full(shape):
    return pl.BlockSpec(shape, lambda p, i: (0,) * len(shape))


def _mlp(x, w1, b1, g1, be1, w2, b2, g2, be2, w3, b3, g3, be3, w4, b4):
    return pl.pallas_call(
        _mlp_body,
        grid=(NPASS, NB),
        in_specs=[
            pl.BlockSpec((BB, INPUT_DIM),
                         lambda p, i: (jnp.where(p == 0, i, 0), 0)),
            _full((INPUT_DIM, 256)), _full((1, 256)), _full((1, 256)), _full((1, 256)),
            _full((256, 256)), _full((1, 256)), _full((1, 256)), _full((1, 256)),
            _full((256, 128)), _full((1, 128)), _full((1, 128)), _full((1, 128)),
            _full((128, 1)), _full((1,)),
        ],
        out_specs=pl.BlockSpec((BB,), lambda p, i: (i,)),
        out_shape=jax.ShapeDtypeStruct((BATCH,), jnp.float32),
        scratch_shapes=[
            pltpu.VMEM((BATCH, 256), jnp.float32),
            pltpu.VMEM((BATCH, 128), jnp.float32),
            pltpu.VMEM((1, 256), jnp.float32),
            pltpu.VMEM((1, 256), jnp.float32),
            pltpu.VMEM((1, 256), jnp.float32),
            pltpu.VMEM((1, 256), jnp.float32),
            pltpu.VMEM((1, 128), jnp.float32),
            pltpu.VMEM((1, 128), jnp.float32),
        ],
        compiler_params=pltpu.CompilerParams(
            vmem_limit_bytes=100 * 1024 * 1024,
        ),
    )(x, w1, b1, g1, be1, w2, b2, g2, be2, w3, b3, g3, be3, w4, b4)


def kernel(features, tables, W1, b1, g1, be1, W2, b2, g2, be2, W3, b3, g3, be3, W4, b4):
    flat_tables = tables.reshape(NUM_FIELDS * VOCAB, EMBED_DIM)
    idx = (features + jnp.arange(NUM_FIELDS, dtype=jnp.int32) * VOCAB)
    idx = idx.reshape(NW, NCHUNK, CHUNK)
    emb = _make_sc_gather()(flat_tables, idx)
    x = emb.reshape(BATCH, INPUT_DIM)
    r = lambda a: a.reshape(1, -1)
    return _mlp(x, W1, r(b1), r(g1), r(be1), W2, r(b2), r(g2), r(be2),
                W3, r(b3), r(g3), r(be3), W4, b4)
